# baseline (speedup 1.0000x reference)
import jax
import jax.numpy as jnp
from jax import lax
from jax.experimental import pallas as pl
from jax.experimental.pallas import tpu as pltpu
from jax.experimental.pallas import tpu_sc as plsc

L = 16384
V = 1000
VA = 504
VB = V - VA
NC, NS, LANES = 2, 16, 16
NW = NC * NS
CPW = L // NW
CCH = 128
NCHUNK = CPW // CCH
VPC = CCH // LANES

SEGS_A = [(0, 64), (64, 128), (192, 152), (344, 160)]
SEGS_B = [(0, 64), (64, 128), (192, 152), (344, 152)]


def _body(x_hbm, out_hbm, idx_v, buf_a, buf_b,
          sem_i, sem_a, sem_b, sems_ra, sems_rb):
    wid = lax.axis_index("s") * NC + lax.axis_index("c")
    base = wid * CPW
    cp_i = pltpu.async_copy(x_hbm.at[pl.ds(base, CPW)], idx_v, sem_i)

    zeros16 = jnp.zeros((LANES,), jnp.float32)
    ones16 = jnp.full((LANES,), 1.0, jnp.float32)

    ZU = 1

    def _zero_rows(buf, lo):
        def body(i, carry):
            for u in range(ZU):
                for off in range(0, CCH, LANES):
                    buf[lo + i * ZU + u, pl.ds(off, LANES)] = zeros16
            return carry
        return body

    def _idx(c, k):
        vv = idx_v[pl.ds(c * CCH + k * LANES, LANES)]
        cols = lax.iota(jnp.int32, LANES) + k * LANES
        return vv, cols

    def _scatter_a(c, val):
        for k in range(VPC):
            vv, cols = _idx(c, k)
            plsc.store_scatter(buf_a, [vv, cols], val, mask=vv < VA)

    def _scatter_b(c, val):
        for k in range(VPC):
            vv, cols = _idx(c, k)
            plsc.store_scatter(buf_b, [vv - VA, cols], val, mask=vv >= VA)

    def _scatter_seg(buf, vbase, lo, n, val):
        glo, ghi = vbase + lo, vbase + lo + n
        for k in range(VPC):
            vv, cols = _idx(0, k)
            m = (vv >= glo) & (vv < ghi)
            plsc.store_scatter(buf, [vv - vbase, cols], val, mask=m)

    def _dma_a(c):
        return pltpu.async_copy(
            buf_a, out_hbm.at[pl.ds(0, VA), pl.ds(base + c * CCH, CCH)], sem_a
        )

    def _dma_b(c):
        return pltpu.async_copy(
            buf_b, out_hbm.at[pl.ds(VA, VB), pl.ds(base + c * CCH, CCH)], sem_b
        )

    ramp = []
    first = True
    for buf, vbase, segs, sems in (
        (buf_a, 0, SEGS_A, sems_ra),
        (buf_b, VA, SEGS_B, sems_rb),
    ):
        for s, (lo, n) in enumerate(segs):
            lax.fori_loop(0, n // ZU, _zero_rows(buf, lo), 0)
            if first:
                cp_i.wait()
                first = False
            _scatter_seg(buf, vbase, lo, n, ones16)
            ramp.append(pltpu.async_copy(
                buf.at[pl.ds(lo, n)],
                out_hbm.at[pl.ds(vbase + lo, n), pl.ds(base, CCH)],
                sems[s],
            ))

    cp_a = cp_b = None
    for c in range(1, NCHUNK):
        if c == 1:
            for cp in ramp[:len(SEGS_A)]:
                cp.wait()
        else:
            cp_a.wait()
        _scatter_a(c - 1, zeros16)
        _scatter_a(c, ones16)
        cp_a = _dma_a(c)
        if c == 1:
            for cp in ramp[len(SEGS_A):]:
                cp.wait()
        else:
            cp_b.wait()
        _scatter_b(c - 1, zeros16)
        _scatter_b(c, ones16)
        cp_b = _dma_b(c)
    cp_a.wait()
    cp_b.wait()


@jax.jit
def _one_hot_sc(xf):
    kfn = pl.kernel(
        _body,
        out_type=jax.ShapeDtypeStruct((V, L), jnp.float32),
        mesh=plsc.VectorSubcoreMesh(core_axis_name="c", subcore_axis_name="s"),
        scratch_types=[
            pltpu.VMEM((CPW,), jnp.int32),
            pltpu.VMEM((VA, CCH), jnp.float32),
            pltpu.VMEM((VB, CCH), jnp.float32),
            pltpu.SemaphoreType.DMA,
            pltpu.SemaphoreType.DMA,
            pltpu.SemaphoreType.DMA,
            [pltpu.SemaphoreType.DMA] * len(SEGS_A),
            [pltpu.SemaphoreType.DMA] * len(SEGS_B),
        ],
        compiler_params=pltpu.CompilerParams(
            use_tc_tiling_on_sc=True,
            needs_layout_passes=False,
            disable_bounds_checks=True,
            disable_semaphore_checks=True,
            skip_device_barrier=True,
        ),
    )
    return kfn(xf)


def kernel(x):
    return _one_hot_sc(x.reshape(L)).T


# speedup vs baseline: 1.0065x; 1.0065x over previous
import jax
import jax.numpy as jnp
from jax import lax
from jax.experimental import pallas as pl
from jax.experimental.pallas import tpu as pltpu
from jax.experimental.pallas import tpu_sc as plsc

L = 16384
V = 1000
VA = 504
VB = V - VA
NC, NS, LANES = 2, 16, 16
NW = NC * NS
CPW = L // NW
CCH = 128
NCHUNK = CPW // CCH
VPC = CCH // LANES

SEGS_A = [(0, 128), (128, 128), (256, 128), (384, 120)]
SEGS_B = [(0, 128), (128, 128), (256, 128), (384, 112)]


def _body(x_hbm, out_hbm, idx_v, buf_a, buf_b,
          sem_i, sem_a, sem_b, sems_ra, sems_rb):
    wid = lax.axis_index("s") * NC + lax.axis_index("c")
    base = wid * CPW
    cp_i = pltpu.async_copy(x_hbm.at[pl.ds(base, CPW)], idx_v, sem_i)

    zeros16 = jnp.zeros((LANES,), jnp.float32)
    ones16 = jnp.full((LANES,), 1.0, jnp.float32)

    ZU = 1

    def _zero_rows(buf, lo):
        def body(i, carry):
            for u in range(ZU):
                for off in range(0, CCH, LANES):
                    buf[lo + i * ZU + u, pl.ds(off, LANES)] = zeros16
            return carry
        return body

    def _idx(c, k):
        vv = idx_v[pl.ds(c * CCH + k * LANES, LANES)]
        cols = lax.iota(jnp.int32, LANES) + k * LANES
        return vv, cols

    def _scatter_a(c, val):
        for k in range(VPC):
            vv, cols = _idx(c, k)
            plsc.store_scatter(buf_a, [vv, cols], val, mask=vv < VA)

    def _scatter_b(c, val):
        for k in range(VPC):
            vv, cols = _idx(c, k)
            plsc.store_scatter(buf_b, [vv - VA, cols], val, mask=vv >= VA)

    def _scatter_seg(buf, vbase, lo, n, val):
        glo, ghi = vbase + lo, vbase + lo + n
        for k in range(VPC):
            vv, cols = _idx(0, k)
            m = (vv >= glo) & (vv < ghi)
            plsc.store_scatter(buf, [vv - vbase, cols], val, mask=m)

    def _dma_a(c):
        return pltpu.async_copy(
            buf_a, out_hbm.at[pl.ds(0, VA), pl.ds(base + c * CCH, CCH)], sem_a
        )

    def _dma_b(c):
        return pltpu.async_copy(
            buf_b, out_hbm.at[pl.ds(VA, VB), pl.ds(base + c * CCH, CCH)], sem_b
        )

    ramp = []
    first = True
    for buf, vbase, segs, sems in (
        (buf_a, 0, SEGS_A, sems_ra),
        (buf_b, VA, SEGS_B, sems_rb),
    ):
        for s, (lo, n) in enumerate(segs):
            lax.fori_loop(0, n // ZU, _zero_rows(buf, lo), 0)
            if first:
                cp_i.wait()
                first = False
            _scatter_seg(buf, vbase, lo, n, ones16)
            ramp.append(pltpu.async_copy(
                buf.at[pl.ds(lo, n)],
                out_hbm.at[pl.ds(vbase + lo, n), pl.ds(base, CCH)],
                sems[s],
            ))

    cp_a = cp_b = None
    for c in range(1, NCHUNK):
        if c == 1:
            for cp in ramp[:len(SEGS_A)]:
                cp.wait()
        else:
            cp_a.wait()
        _scatter_a(c - 1, zeros16)
        _scatter_a(c, ones16)
        cp_a = _dma_a(c)
        if c == 1:
            for cp in ramp[len(SEGS_A):]:
                cp.wait()
        else:
            cp_b.wait()
        _scatter_b(c - 1, zeros16)
        _scatter_b(c, ones16)
        cp_b = _dma_b(c)
    cp_a.wait()
    cp_b.wait()


@jax.jit
def _one_hot_sc(xf):
    kfn = pl.kernel(
        _body,
        out_type=jax.ShapeDtypeStruct((V, L), jnp.float32),
        mesh=plsc.VectorSubcoreMesh(core_axis_name="c", subcore_axis_name="s"),
        scratch_types=[
            pltpu.VMEM((CPW,), jnp.int32),
            pltpu.VMEM((VA, CCH), jnp.float32),
            pltpu.VMEM((VB, CCH), jnp.float32),
            pltpu.SemaphoreType.DMA,
            pltpu.SemaphoreType.DMA,
            pltpu.SemaphoreType.DMA,
            [pltpu.SemaphoreType.DMA] * len(SEGS_A),
            [pltpu.SemaphoreType.DMA] * len(SEGS_B),
        ],
        compiler_params=pltpu.CompilerParams(
            use_tc_tiling_on_sc=True,
            needs_layout_passes=False,
            disable_bounds_checks=True,
            disable_semaphore_checks=True,
            skip_device_barrier=True,
        ),
    )
    return kfn(xf)


def kernel(x):
    return _one_hot_sc(x.reshape(L)).T
